# baseline (speedup 1.0000x reference)
import functools

import jax
import jax.numpy as jnp
from jax import lax
from jax.experimental import pallas as pl
from jax.experimental.pallas import tpu as pltpu
from jax.experimental.pallas import tpu_sc as plsc

N = 10000
E = 160000
DIN = 128
DE = 16
H = 32
EH = 128
HH = H * H
N_MP = 3
N_S2S = 6

NC = 2
NS = 16
NW = NC * NS
EPW = 5120
E_PAD = NW * EPW
CHUNK = 1024
NCHUNK = EPW // CHUNK
GROUPS = CHUNK // 128
IDX_ROWS = E_PAD // 128
NPAD = 10240
RPT = NPAD // NS

BE = 512


def _proj_body(x_ref, w_ref, b_ref, o_ref):
    o_ref[...] = jnp.maximum(
        jnp.dot(x_ref[...], w_ref[...], preferred_element_type=jnp.float32, precision=jax.lax.Precision.HIGHEST)
        + b_ref[...], 0.0)


def _split3(x):
    hi = x.astype(jnp.bfloat16)
    r = x - hi.astype(jnp.float32)
    mid = r.astype(jnp.bfloat16)
    lo = (r - mid.astype(jnp.float32)).astype(jnp.bfloat16)
    return hi, mid, lo


def _ehid_body(w1t_ref, b1_ref, ef_ref, o_ref):
    ehid = jnp.maximum(
        jnp.dot(w1t_ref[...], ef_ref[...], preferred_element_type=jnp.float32, precision=jax.lax.Precision.HIGHEST)
        + b1_ref[...], 0.0)
    bh, bm, bl = _split3(ehid)
    o_ref[...] = jnp.concatenate([bh, bm, bl], axis=0)


def _msg_body(w2cat_ref, b2_ref, ehcat_ref, hs_ref, o_ref):
    b_full = ehcat_ref[...]
    bcat = jnp.concatenate(
        [b_full, b_full[0:2 * EH], b_full[0:EH]], axis=0)
    we = lax.dot_general(
        w2cat_ref[...], bcat, (((1,), (0,)), ((), ())),
        preferred_element_type=jnp.float32) + b2_ref[...]
    hst = hs_ref[...].T
    acc = we[0:H] * hst[0:1]
    for hh in range(1, H):
        acc = acc + we[H * hh:H * (hh + 1)] * hst[hh:hh + 1]
    o_ref[...] = acc.T


def _gru_body(agg_ref, h_ref, cb_ref, wir_ref, whr_ref, br_ref, wiz_ref,
              whz_ref, bz_ref, win_ref, bin_ref, whn_ref, bhn_ref, o_ref):
    a = agg_ref[0] + agg_ref[1] + cb_ref[...]
    m = jnp.maximum(a, 0.0)
    h = h_ref[...]

    def mm(x, w):
        return jnp.dot(x, w[...], preferred_element_type=jnp.float32, precision=jax.lax.Precision.HIGHEST)

    r = jax.nn.sigmoid(mm(m, wir_ref) + mm(h, whr_ref) + br_ref[...])
    z = jax.nn.sigmoid(mm(m, wiz_ref) + mm(h, whz_ref) + bz_ref[...])
    n = jnp.tanh(mm(m, win_ref) + bin_ref[...]
                 + r * (mm(h, whn_ref) + bhn_ref[...]))
    o_ref[...] = (1.0 - z) * n + z * h


def _s2s_body(feat_ref,
              wih0_ref, whh0_ref, b0_ref,
              wih1_ref, whh1_ref, b1_ref,
              wih2_ref, whh2_ref, b2_ref,
              procw_ref, procb_ref,
              p1_ref, pb1_ref, p2_ref, pb2_ref, p3_ref, pb3_ref,
              o_ref):
    feat = feat_ref[...]

    def mm(x, w):
        return jnp.dot(x, w[...], preferred_element_type=jnp.float32, precision=jax.lax.Precision.HIGHEST)

    hs = [jnp.zeros((1, H), jnp.float32) for _ in range(3)]
    cs = [jnp.zeros((1, H), jnp.float32) for _ in range(3)]
    lw = [(wih0_ref, whh0_ref, b0_ref), (wih1_ref, whh1_ref, b1_ref),
          (wih2_ref, whh2_ref, b2_ref)]
    q_star = jnp.zeros((1, 2 * H), jnp.float32)
    for _ in range(N_S2S):
        x = q_star
        for l in range(3):
            wih, whh, b = lw[l]
            g = mm(x, wih) + mm(hs[l], whh) + b[...]
            i = jax.nn.sigmoid(g[:, 0:H])
            f = jax.nn.sigmoid(g[:, H:2 * H])
            gg = jnp.tanh(g[:, 2 * H:3 * H])
            o = jax.nn.sigmoid(g[:, 3 * H:4 * H])
            cs[l] = f * cs[l] + i * gg
            hs[l] = o * jnp.tanh(cs[l])
            x = hs[l]
        q = x
        e = jnp.sum(feat * q, axis=1, keepdims=True)
        emax = jnp.max(e, axis=0, keepdims=True)
        ex = jnp.exp(e - emax)
        es = jnp.sum(ex, axis=0, keepdims=True)
        alpha = ex / es
        readout = jnp.sum(feat * alpha, axis=0, keepdims=True)
        q_star = jnp.concatenate([q, readout], axis=1)
    g_feat = jnp.maximum(mm(q_star, procw_ref) + procb_ref[...], 0.0)
    x = jnp.maximum(mm(g_feat, p1_ref) + pb1_ref[...], 0.0)
    x = jnp.maximum(mm(x, p2_ref) + pb2_ref[...], 0.0)
    o_ref[...] = mm(x, p3_ref) + pb3_ref[...]


_SC_KERNELS = {}


def _get_sc_kernels():
    if _SC_KERNELS:
        return _SC_KERNELS["gather"], _SC_KERNELS["scatter"]

    mesh = plsc.VectorSubcoreMesh(core_axis_name="c", subcore_axis_name="s",
                                  num_cores=NC, num_subcores=NS)

    @functools.partial(
        pl.kernel,
        out_type=jax.ShapeDtypeStruct((E_PAD, H), jnp.float32),
        mesh=mesh,
        compiler_params=pltpu.CompilerParams(use_tc_tiling_on_sc=False),
        scratch_types=[
            pltpu.VMEM((GROUPS, 128), jnp.int32),
            pltpu.VMEM((CHUNK, H), jnp.float32),
            pltpu.SemaphoreType.DMA,
        ],
    )
    def _sc_gather(h_hbm, idx_hbm, out_hbm, idx_v, rows_v, sem):
        c = lax.axis_index("c")
        s = lax.axis_index("s")
        wid = s * NC + c

        def chunk_body(k, carry):
            base_row = wid * (EPW // 128) + k * GROUPS
            pltpu.sync_copy(idx_hbm.at[pl.ds(base_row, GROUPS)], idx_v)
            copies = []
            for j in range(GROUPS):
                copies.append(pltpu.async_copy(
                    h_hbm.at[idx_v.at[j]],
                    rows_v.at[pl.ds(j * 128, 128)], sem))
            for d in copies:
                d.wait()
            pltpu.sync_copy(rows_v,
                            out_hbm.at[pl.ds(wid * EPW + k * CHUNK, CHUNK)])
            return carry

        lax.fori_loop(0, NCHUNK, chunk_body, 0)

    @functools.partial(
        pl.kernel,
        out_type=jax.ShapeDtypeStruct((NC, NPAD, H), jnp.float32),
        mesh=mesh,
        compiler_params=pltpu.CompilerParams(use_tc_tiling_on_sc=False),
        scratch_types=[
            pltpu.VMEM((GROUPS, 128), jnp.int32),
            pltpu.VMEM((CHUNK, H), jnp.float32),
            pltpu.VMEM((RPT, H), jnp.float32),
            pltpu.VMEM_SHARED((NPAD, H), jnp.float32),
            pltpu.SemaphoreType.DMA,
        ],
    )
    def _sc_scatter(msg_hbm, dst_hbm, zeros_hbm, out_hbm, idx_v, msg_v,
                    tile_v, agg_sh, sem):
        c = lax.axis_index("c")
        s = lax.axis_index("s")
        wid = s * NC + c
        pltpu.sync_copy(zeros_hbm, tile_v)
        pltpu.sync_copy(tile_v, agg_sh.at[pl.ds(s * RPT, RPT)])
        plsc.subcore_barrier()

        def chunk_body(k, carry):
            base_row = wid * (EPW // 128) + k * GROUPS
            pltpu.sync_copy(dst_hbm.at[pl.ds(base_row, GROUPS)], idx_v)
            pltpu.sync_copy(msg_hbm.at[pl.ds(wid * EPW + k * CHUNK, CHUNK)],
                            msg_v)
            for j in range(GROUPS):
                pltpu.sync_copy(msg_v.at[pl.ds(j * 128, 128)],
                                agg_sh.at[idx_v.at[j]], add=True)
            return carry

        lax.fori_loop(0, NCHUNK, chunk_body, 0)
        plsc.subcore_barrier()
        pltpu.sync_copy(agg_sh.at[pl.ds(s * RPT, RPT)],
                        out_hbm.at[c, pl.ds(s * RPT, RPT)])

    _SC_KERNELS["gather"] = _sc_gather
    _SC_KERNELS["scatter"] = _sc_scatter
    return _sc_gather, _sc_scatter


def kernel(node_feats, edge_feats, edge_index, proj_W, proj_b, eW1, eb1, eW2,
           eb2, conv_b, gru_Wih, gru_Whh, gru_bih, gru_bhh, lstm_Wih0,
           lstm_Whh0, lstm_bih0, lstm_bhh0, lstm_Wih1, lstm_Whh1, lstm_bih1,
           lstm_bhh1, lstm_Wih2, lstm_Whh2, lstm_bih2, lstm_bhh2, proc_W,
           proc_b, pred_W1, pred_b1, pred_W2, pred_b2, pred_W3, pred_b3):
    f32 = jnp.float32

    src2d = jnp.pad(edge_index[0], (0, E_PAD - E)).reshape(IDX_ROWS, 128)
    dst2d = jnp.pad(edge_index[1], (0, E_PAD - E),
                    constant_values=N).reshape(IDX_ROWS, 128)
    ef_t = jnp.pad(edge_feats, ((0, E_PAD - E), (0, 0))).T
    w1t = eW1.T
    b1c = eb1.reshape(EH, 1)
    w2t = eW2.T
    b2c = eb2.reshape(HH, 1)
    zeros_rpt = jnp.zeros((RPT, H), f32)

    wir, wiz, win = gru_Wih[0:H].T, gru_Wih[H:2 * H].T, gru_Wih[2 * H:].T
    whr, whz, whn = gru_Whh[0:H].T, gru_Whh[H:2 * H].T, gru_Whh[2 * H:].T
    br = (gru_bih[0:H] + gru_bhh[0:H]).reshape(1, H)
    bz = (gru_bih[H:2 * H] + gru_bhh[H:2 * H]).reshape(1, H)
    bin_ = gru_bih[2 * H:].reshape(1, H)
    bhn = gru_bhh[2 * H:].reshape(1, H)
    cb = conv_b.reshape(1, H)

    lstm = [
        (lstm_Wih0.T, lstm_Whh0.T, (lstm_bih0 + lstm_bhh0).reshape(1, 4 * H)),
        (lstm_Wih1.T, lstm_Whh1.T, (lstm_bih1 + lstm_bhh1).reshape(1, 4 * H)),
        (lstm_Wih2.T, lstm_Whh2.T, (lstm_bih2 + lstm_bhh2).reshape(1, 4 * H)),
    ]

    BN = 2000
    h = pl.pallas_call(
        _proj_body,
        grid=(N // BN,),
        in_specs=[
            pl.BlockSpec((BN, DIN), lambda i: (i, 0)),
            pl.BlockSpec((DIN, H), lambda i: (0, 0)),
            pl.BlockSpec((1, H), lambda i: (0, 0)),
        ],
        out_specs=pl.BlockSpec((BN, H), lambda i: (i, 0)),
        out_shape=jax.ShapeDtypeStruct((N, H), f32),
    )(node_feats, proj_W, proj_b.reshape(1, H))

    BE2 = 2048
    ehcat = pl.pallas_call(
        _ehid_body,
        grid=(E_PAD // BE2,),
        in_specs=[
            pl.BlockSpec((EH, DE), lambda i: (0, 0)),
            pl.BlockSpec((EH, 1), lambda i: (0, 0)),
            pl.BlockSpec((DE, BE2), lambda i: (0, i)),
        ],
        out_specs=pl.BlockSpec((3 * EH, BE2), lambda i: (0, i)),
        out_shape=jax.ShapeDtypeStruct((3 * EH, E_PAD), jnp.bfloat16),
    )(w1t, b1c, ef_t)
    ah = w2t.astype(jnp.bfloat16)
    _r = w2t - ah.astype(f32)
    am = _r.astype(jnp.bfloat16)
    al = (_r - am.astype(f32)).astype(jnp.bfloat16)
    w2cat = jnp.concatenate([ah, ah, ah, am, am, al], axis=1)

    _sc_gather, _sc_scatter = _get_sc_kernels()

    hidden = h
    for _ in range(N_MP):
        h_src = _sc_gather(hidden, src2d)
        msg = pl.pallas_call(
            _msg_body,
            grid=(E_PAD // BE,),
            in_specs=[
                pl.BlockSpec((HH, 6 * EH), lambda i: (0, 0)),
                pl.BlockSpec((HH, 1), lambda i: (0, 0)),
                pl.BlockSpec((3 * EH, BE), lambda i: (0, i)),
                pl.BlockSpec((BE, H), lambda i: (i, 0)),
            ],
            out_specs=pl.BlockSpec((BE, H), lambda i: (i, 0)),
            out_shape=jax.ShapeDtypeStruct((E_PAD, H), f32),
        )(w2cat, b2c, ehcat, h_src)
        agg2 = _sc_scatter(msg, dst2d, zeros_rpt)
        wspec = pl.BlockSpec((H, H), lambda i: (0, 0))
        bspec = pl.BlockSpec((1, H), lambda i: (0, 0))
        hidden = pl.pallas_call(
            _gru_body,
            grid=(N // BN,),
            in_specs=[
                pl.BlockSpec((NC, BN, H), lambda i: (0, i, 0)),
                pl.BlockSpec((BN, H), lambda i: (i, 0)),
                bspec, wspec, wspec, bspec, wspec, wspec, bspec, wspec,
                bspec, wspec, bspec,
            ],
            out_specs=pl.BlockSpec((BN, H), lambda i: (i, 0)),
            out_shape=jax.ShapeDtypeStruct((N, H), f32),
        )(agg2, hidden, cb, wir, whr, br, wiz, whz, bz, win, bin_, whn, bhn)

    out = pl.pallas_call(
        _s2s_body,
        out_shape=jax.ShapeDtypeStruct((1, 1), f32),
    )(hidden,
      lstm[0][0], lstm[0][1], lstm[0][2],
      lstm[1][0], lstm[1][1], lstm[1][2],
      lstm[2][0], lstm[2][1], lstm[2][2],
      proc_W, proc_b.reshape(1, H),
      pred_W1, pred_b1.reshape(1, H), pred_W2, pred_b2.reshape(1, H),
      pred_W3, pred_b3.reshape(1, 1))
    return out


# speedup vs baseline: 1.1122x; 1.1122x over previous
import functools

import jax
import jax.numpy as jnp
from jax import lax
from jax.experimental import pallas as pl
from jax.experimental.pallas import tpu as pltpu
from jax.experimental.pallas import tpu_sc as plsc

N = 10000
E = 160000
DIN = 128
DE = 16
H = 32
EH = 128
HH = H * H
N_MP = 3
N_S2S = 6

NC = 2
NS = 16
NW = NC * NS
EPW = 5120
E_PAD = NW * EPW
CHUNK = 1024
NCHUNK = EPW // CHUNK
GROUPS = CHUNK // 128
IDX_ROWS = E_PAD // 128
NPAD = 10240
RPT = NPAD // NS

BE = 1024


def _proj_body(x_ref, w_ref, b_ref, o_ref):
    o_ref[...] = jnp.maximum(
        jnp.dot(x_ref[...], w_ref[...], preferred_element_type=jnp.float32, precision=jax.lax.Precision.HIGHEST)
        + b_ref[...], 0.0)


def _split3(x):
    hi = x.astype(jnp.bfloat16)
    r = x - hi.astype(jnp.float32)
    mid = r.astype(jnp.bfloat16)
    lo = (r - mid.astype(jnp.float32)).astype(jnp.bfloat16)
    return hi, mid, lo


def _ehid_body(w1t_ref, b1_ref, ef_ref, o_ref):
    ehid = jnp.maximum(
        jnp.dot(w1t_ref[...], ef_ref[...], preferred_element_type=jnp.float32, precision=jax.lax.Precision.HIGHEST)
        + b1_ref[...], 0.0)
    bh, bm, bl = _split3(ehid)
    o_ref[...] = jnp.concatenate([bh, bm, bl], axis=0)


def _msg_body(w2cat_ref, b2_ref, ehcat_ref, hs_ref, o_ref):
    b_full = ehcat_ref[...]
    bcat = jnp.concatenate(
        [b_full, b_full[0:2 * EH], b_full[0:EH]], axis=0)
    we = lax.dot_general(
        w2cat_ref[...], bcat, (((1,), (0,)), ((), ())),
        preferred_element_type=jnp.float32) + b2_ref[...]
    hst = hs_ref[...].T
    acc = we[0:H] * hst[0:1]
    for hh in range(1, H):
        acc = acc + we[H * hh:H * (hh + 1)] * hst[hh:hh + 1]
    o_ref[...] = acc.T


def _gru_body(agg_ref, h_ref, cb_ref, wir_ref, whr_ref, br_ref, wiz_ref,
              whz_ref, bz_ref, win_ref, bin_ref, whn_ref, bhn_ref, o_ref):
    a = agg_ref[0] + agg_ref[1] + cb_ref[...]
    m = jnp.maximum(a, 0.0)
    h = h_ref[...]

    def mm(x, w):
        return jnp.dot(x, w[...], preferred_element_type=jnp.float32, precision=jax.lax.Precision.HIGHEST)

    r = jax.nn.sigmoid(mm(m, wir_ref) + mm(h, whr_ref) + br_ref[...])
    z = jax.nn.sigmoid(mm(m, wiz_ref) + mm(h, whz_ref) + bz_ref[...])
    n = jnp.tanh(mm(m, win_ref) + bin_ref[...]
                 + r * (mm(h, whn_ref) + bhn_ref[...]))
    o_ref[...] = (1.0 - z) * n + z * h


def _s2s_body(feat_ref,
              wih0_ref, whh0_ref, b0_ref,
              wih1_ref, whh1_ref, b1_ref,
              wih2_ref, whh2_ref, b2_ref,
              procw_ref, procb_ref,
              p1_ref, pb1_ref, p2_ref, pb2_ref, p3_ref, pb3_ref,
              o_ref):
    feat = feat_ref[...]

    def mm(x, w):
        return jnp.dot(x, w[...], preferred_element_type=jnp.float32, precision=jax.lax.Precision.HIGHEST)

    hs = [jnp.zeros((1, H), jnp.float32) for _ in range(3)]
    cs = [jnp.zeros((1, H), jnp.float32) for _ in range(3)]
    lw = [(wih0_ref, whh0_ref, b0_ref), (wih1_ref, whh1_ref, b1_ref),
          (wih2_ref, whh2_ref, b2_ref)]
    q_star = jnp.zeros((1, 2 * H), jnp.float32)
    for _ in range(N_S2S):
        x = q_star
        for l in range(3):
            wih, whh, b = lw[l]
            g = mm(x, wih) + mm(hs[l], whh) + b[...]
            i = jax.nn.sigmoid(g[:, 0:H])
            f = jax.nn.sigmoid(g[:, H:2 * H])
            gg = jnp.tanh(g[:, 2 * H:3 * H])
            o = jax.nn.sigmoid(g[:, 3 * H:4 * H])
            cs[l] = f * cs[l] + i * gg
            hs[l] = o * jnp.tanh(cs[l])
            x = hs[l]
        q = x
        e = jnp.sum(feat * q, axis=1, keepdims=True)
        emax = jnp.max(e, axis=0, keepdims=True)
        ex = jnp.exp(e - emax)
        es = jnp.sum(ex, axis=0, keepdims=True)
        alpha = ex / es
        readout = jnp.sum(feat * alpha, axis=0, keepdims=True)
        q_star = jnp.concatenate([q, readout], axis=1)
    g_feat = jnp.maximum(mm(q_star, procw_ref) + procb_ref[...], 0.0)
    x = jnp.maximum(mm(g_feat, p1_ref) + pb1_ref[...], 0.0)
    x = jnp.maximum(mm(x, p2_ref) + pb2_ref[...], 0.0)
    o_ref[...] = mm(x, p3_ref) + pb3_ref[...]


_SC_KERNELS = {}


def _get_sc_kernels():
    if _SC_KERNELS:
        return _SC_KERNELS["gather"], _SC_KERNELS["scatter"]

    mesh = plsc.VectorSubcoreMesh(core_axis_name="c", subcore_axis_name="s",
                                  num_cores=NC, num_subcores=NS)

    @functools.partial(
        pl.kernel,
        out_type=jax.ShapeDtypeStruct((E_PAD, H), jnp.float32),
        mesh=mesh,
        compiler_params=pltpu.CompilerParams(use_tc_tiling_on_sc=False),
        scratch_types=[
            pltpu.VMEM((NCHUNK, GROUPS, 128), jnp.int32),
            pltpu.VMEM((2, CHUNK, H), jnp.float32),
            pltpu.SemaphoreType.DMA,
            pltpu.SemaphoreType.DMA,
            pltpu.SemaphoreType.DMA,
        ],
    )
    def _sc_gather(h_hbm, idx_hbm, out_hbm, idx_v, rows_v, gsem, wsem,
                   isem):
        c = lax.axis_index("c")
        s = lax.axis_index("s")
        wid = s * NC + c
        idx_loads = []
        gathers = [None] * NCHUNK
        for k in range(NCHUNK):
            base_row = wid * (EPW // 128) + k * GROUPS
            idx_loads.append(pltpu.async_copy(
                idx_hbm.at[pl.ds(base_row, GROUPS)], idx_v.at[k], isem))
        for k in range(NCHUNK):
            if k >= 2:
                for d in gathers[k - 2]:
                    d.wait()
                pltpu.async_copy(
                    rows_v.at[k % 2],
                    out_hbm.at[pl.ds(wid * EPW + (k - 2) * CHUNK, CHUNK)],
                    wsem).wait()
            idx_loads[k].wait()
            gathers[k] = [
                pltpu.async_copy(
                    h_hbm.at[idx_v.at[k, j]],
                    rows_v.at[k % 2, pl.ds(j * 128, 128)], gsem)
                for j in range(GROUPS)]
        for k in range(max(NCHUNK - 2, 0), NCHUNK):
            for d in gathers[k]:
                d.wait()
            pltpu.sync_copy(rows_v.at[k % 2],
                            out_hbm.at[pl.ds(wid * EPW + k * CHUNK, CHUNK)])

    @functools.partial(
        pl.kernel,
        out_type=jax.ShapeDtypeStruct((NC, NPAD, H), jnp.float32),
        mesh=mesh,
        compiler_params=pltpu.CompilerParams(use_tc_tiling_on_sc=False),
        scratch_types=[
            pltpu.VMEM((GROUPS, 128), jnp.int32),
            pltpu.VMEM((CHUNK, H), jnp.float32),
            pltpu.VMEM((RPT, H), jnp.float32),
            pltpu.VMEM_SHARED((NPAD, H), jnp.float32),
            pltpu.SemaphoreType.DMA,
        ],
    )
    def _sc_scatter(msg_hbm, dst_hbm, zeros_hbm, out_hbm, idx_v, msg_v,
                    tile_v, agg_sh, sem):
        c = lax.axis_index("c")
        s = lax.axis_index("s")
        wid = s * NC + c
        pltpu.sync_copy(zeros_hbm, tile_v)
        pltpu.sync_copy(tile_v, agg_sh.at[pl.ds(s * RPT, RPT)])
        plsc.subcore_barrier()

        def chunk_body(k, carry):
            base_row = wid * (EPW // 128) + k * GROUPS
            pltpu.sync_copy(dst_hbm.at[pl.ds(base_row, GROUPS)], idx_v)
            pltpu.sync_copy(msg_hbm.at[pl.ds(wid * EPW + k * CHUNK, CHUNK)],
                            msg_v)
            for j in range(GROUPS):
                pltpu.sync_copy(msg_v.at[pl.ds(j * 128, 128)],
                                agg_sh.at[idx_v.at[j]], add=True)
            return carry

        lax.fori_loop(0, NCHUNK, chunk_body, 0)
        plsc.subcore_barrier()
        pltpu.sync_copy(agg_sh.at[pl.ds(s * RPT, RPT)],
                        out_hbm.at[c, pl.ds(s * RPT, RPT)])

    _SC_KERNELS["gather"] = _sc_gather
    _SC_KERNELS["scatter"] = _sc_scatter
    return _sc_gather, _sc_scatter


def kernel(node_feats, edge_feats, edge_index, proj_W, proj_b, eW1, eb1, eW2,
           eb2, conv_b, gru_Wih, gru_Whh, gru_bih, gru_bhh, lstm_Wih0,
           lstm_Whh0, lstm_bih0, lstm_bhh0, lstm_Wih1, lstm_Whh1, lstm_bih1,
           lstm_bhh1, lstm_Wih2, lstm_Whh2, lstm_bih2, lstm_bhh2, proc_W,
           proc_b, pred_W1, pred_b1, pred_W2, pred_b2, pred_W3, pred_b3):
    f32 = jnp.float32

    src2d = jnp.pad(edge_index[0], (0, E_PAD - E)).reshape(IDX_ROWS, 128)
    dst2d = jnp.pad(edge_index[1], (0, E_PAD - E),
                    constant_values=N).reshape(IDX_ROWS, 128)
    ef_t = jnp.pad(edge_feats, ((0, E_PAD - E), (0, 0))).T
    w1t = eW1.T
    b1c = eb1.reshape(EH, 1)
    w2t = eW2.T
    b2c = eb2.reshape(HH, 1)
    zeros_rpt = jnp.zeros((RPT, H), f32)

    wir, wiz, win = gru_Wih[0:H].T, gru_Wih[H:2 * H].T, gru_Wih[2 * H:].T
    whr, whz, whn = gru_Whh[0:H].T, gru_Whh[H:2 * H].T, gru_Whh[2 * H:].T
    br = (gru_bih[0:H] + gru_bhh[0:H]).reshape(1, H)
    bz = (gru_bih[H:2 * H] + gru_bhh[H:2 * H]).reshape(1, H)
    bin_ = gru_bih[2 * H:].reshape(1, H)
    bhn = gru_bhh[2 * H:].reshape(1, H)
    cb = conv_b.reshape(1, H)

    lstm = [
        (lstm_Wih0.T, lstm_Whh0.T, (lstm_bih0 + lstm_bhh0).reshape(1, 4 * H)),
        (lstm_Wih1.T, lstm_Whh1.T, (lstm_bih1 + lstm_bhh1).reshape(1, 4 * H)),
        (lstm_Wih2.T, lstm_Whh2.T, (lstm_bih2 + lstm_bhh2).reshape(1, 4 * H)),
    ]

    BN = 2000
    h = pl.pallas_call(
        _proj_body,
        grid=(N // BN,),
        in_specs=[
            pl.BlockSpec((BN, DIN), lambda i: (i, 0)),
            pl.BlockSpec((DIN, H), lambda i: (0, 0)),
            pl.BlockSpec((1, H), lambda i: (0, 0)),
        ],
        out_specs=pl.BlockSpec((BN, H), lambda i: (i, 0)),
        out_shape=jax.ShapeDtypeStruct((N, H), f32),
    )(node_feats, proj_W, proj_b.reshape(1, H))

    BE2 = 2048
    ehcat = pl.pallas_call(
        _ehid_body,
        grid=(E_PAD // BE2,),
        in_specs=[
            pl.BlockSpec((EH, DE), lambda i: (0, 0)),
            pl.BlockSpec((EH, 1), lambda i: (0, 0)),
            pl.BlockSpec((DE, BE2), lambda i: (0, i)),
        ],
        out_specs=pl.BlockSpec((3 * EH, BE2), lambda i: (0, i)),
        out_shape=jax.ShapeDtypeStruct((3 * EH, E_PAD), jnp.bfloat16),
    )(w1t, b1c, ef_t)
    ah = w2t.astype(jnp.bfloat16)
    _r = w2t - ah.astype(f32)
    am = _r.astype(jnp.bfloat16)
    al = (_r - am.astype(f32)).astype(jnp.bfloat16)
    w2cat = jnp.concatenate([ah, ah, ah, am, am, al], axis=1)

    _sc_gather, _sc_scatter = _get_sc_kernels()

    hidden = h
    for _ in range(N_MP):
        h_src = _sc_gather(hidden, src2d)
        msg = pl.pallas_call(
            _msg_body,
            grid=(E_PAD // BE,),
            in_specs=[
                pl.BlockSpec((HH, 6 * EH), lambda i: (0, 0)),
                pl.BlockSpec((HH, 1), lambda i: (0, 0)),
                pl.BlockSpec((3 * EH, BE), lambda i: (0, i)),
                pl.BlockSpec((BE, H), lambda i: (i, 0)),
            ],
            out_specs=pl.BlockSpec((BE, H), lambda i: (i, 0)),
            out_shape=jax.ShapeDtypeStruct((E_PAD, H), f32),
        )(w2cat, b2c, ehcat, h_src)
        agg2 = _sc_scatter(msg, dst2d, zeros_rpt)
        wspec = pl.BlockSpec((H, H), lambda i: (0, 0))
        bspec = pl.BlockSpec((1, H), lambda i: (0, 0))
        hidden = pl.pallas_call(
            _gru_body,
            grid=(N // BN,),
            in_specs=[
                pl.BlockSpec((NC, BN, H), lambda i: (0, i, 0)),
                pl.BlockSpec((BN, H), lambda i: (i, 0)),
                bspec, wspec, wspec, bspec, wspec, wspec, bspec, wspec,
                bspec, wspec, bspec,
            ],
            out_specs=pl.BlockSpec((BN, H), lambda i: (i, 0)),
            out_shape=jax.ShapeDtypeStruct((N, H), f32),
        )(agg2, hidden, cb, wir, whr, br, wiz, whz, bz, win, bin_, whn, bhn)

    out = pl.pallas_call(
        _s2s_body,
        out_shape=jax.ShapeDtypeStruct((1, 1), f32),
    )(hidden,
      lstm[0][0], lstm[0][1], lstm[0][2],
      lstm[1][0], lstm[1][1], lstm[1][2],
      lstm[2][0], lstm[2][1], lstm[2][2],
      proc_W, proc_b.reshape(1, H),
      pred_W1, pred_b1.reshape(1, H), pred_W2, pred_b2.reshape(1, H),
      pred_W3, pred_b3.reshape(1, 1))
    return out
